# baseline (speedup 1.0000x reference)
import functools

import jax
import jax.numpy as jnp
from jax import lax
from jax.experimental import pallas as pl
from jax.experimental.pallas import tpu as pltpu
from jax.experimental.pallas import tpu_sc as plsc

N_PAD = 96
N_EDGE_CH = 96
CH_PT = N_EDGE_CH
NUM_CORES = 2
NUM_SUB = 16
COLS_PT = 256
OFFS = tuple(range(0, COLS_PT, 16))


def _tanh(x):
    ax = jnp.abs(x)
    t = 1.0 - 2.0 / (jnp.exp(2.0 * ax) + 1.0)
    return jnp.sign(x) * t


def _rsqrt(d):
    bits = lax.bitcast_convert_type(d, jnp.int32)
    y = lax.bitcast_convert_type(
        jnp.int32(0x5F3759DF) - (bits >> 1), jnp.float32)
    half = 0.5 * d
    for _ in range(4):
        y = y * (1.5 - half * y * y)
    return y


def _sc_gcn(ed, fp, wfc, bfc):
    mesh = plsc.VectorSubcoreMesh(
        core_axis_name="c", subcore_axis_name="s", num_cores=NUM_CORES)

    @functools.partial(
        pl.kernel,
        mesh=mesh,
        out_type=jax.ShapeDtypeStruct((6400,), jnp.float32),
        compiler_params=pltpu.CompilerParams(needs_layout_passes=False),
        scratch_types=[
            pltpu.VMEM((3072,), jnp.int32),
            pltpu.VMEM((112,), jnp.float32),
            pltpu.VMEM((94, COLS_PT), jnp.float32),
            pltpu.VMEM((COLS_PT,), jnp.float32),
            pltpu.VMEM((N_PAD,), jnp.float32),
            pltpu.VMEM((NUM_SUB, N_PAD), jnp.float32),
            pltpu.VMEM((N_PAD,), jnp.float32),
            pltpu.VMEM((N_PAD,), jnp.float32),
            pltpu.VMEM((N_PAD,), jnp.float32),
            pltpu.VMEM((N_PAD,), jnp.float32),
            pltpu.VMEM_SHARED((NUM_SUB, N_PAD), jnp.float32),
            pltpu.VMEM_SHARED((NUM_SUB, N_PAD), jnp.float32),
            pltpu.VMEM_SHARED((NUM_SUB, N_PAD), jnp.float32),
            pltpu.SemaphoreType.DMA,
            pltpu.SemaphoreType.DMA,
        ],
    )
    def k(ed_hbm, fp_hbm, wfc_hbm, bfc_hbm, out_hbm,
          ed_v, fp_v, wblk_v, obuf_v, part_v, rb_v, dinv_v,
          g1_v, g2_v, v_v, sh_deg, sh_l1, sh_l2, wsem, ssem):
        sid = lax.axis_index("s")
        wid = sid * NUM_CORES + lax.axis_index("c")
        base = ((wid * 50) // 32) * 128
        ebase = sid * (CH_PT * 16)
        ebase = ebase * 0

        wcp = pltpu.make_async_copy(
            wfc_hbm.at[:, pl.ds(base, COLS_PT)], wblk_v, wsem)
        wcp.start()
        cps = [
            pltpu.make_async_copy(ed_hbm, ed_v, ssem),
            pltpu.make_async_copy(fp_hbm, fp_v, ssem),
            pltpu.make_async_copy(
                bfc_hbm.at[pl.ds(base, COLS_PT)], obuf_v, ssem),
        ]
        for cp in cps:
            cp.start()
        for cp in cps:
            cp.wait()

        zeros = jnp.zeros((16,), jnp.float32)
        ones = jnp.ones((16,), jnp.float32)

        def zero_part():
            for i in range(N_PAD // 16):
                part_v[pl.ds(i * 16, 16)] = zeros

        def reduce_partials(shared, extra):
            pltpu.sync_copy(part_v, shared.at[sid])
            plsc.subcore_barrier()
            for r in range(NUM_SUB):
                pltpu.sync_copy(shared.at[r], rb_v.at[r])
            out = []
            for i in range(N_PAD // 16):
                sl = pl.ds(i * 16, 16)
                acc = rb_v[0, sl]
                for r in range(1, NUM_SUB):
                    acc = acc + rb_v[r, sl]
                out.append(extra(i) + acc * (1.0 / 16.0))
            return out

        zero_part()
        for j in range(CH_PT):
            c = ed_v[pl.ds(1536 + ebase + j * 16, 16)]
            plsc.addupdate_scatter(part_v, [c], ones)
        degs = reduce_partials(sh_deg, lambda i: ones)

        pv = fp_v[pl.ds(96, 16)]
        w10, w11, w12, w13 = pv[0], pv[1], pv[2], pv[3]
        b10, b11, b12, b13 = pv[4], pv[5], pv[6], pv[7]
        w20, w21, w22, w23 = pv[8], pv[9], pv[10], pv[11]
        b2s = pv[12]

        for i in range(N_PAD // 16):
            sl = pl.ds(i * 16, 16)
            di = _rsqrt(degs[i])
            dinv_v[sl] = di
            g1_v[sl] = di * fp_v[sl]

        zero_part()
        for j in range(CH_PT):
            r = ed_v[pl.ds(ebase + j * 16, 16)]
            c = ed_v[pl.ds(1536 + ebase + j * 16, 16)]
            g = plsc.load_gather(g1_v, [r])
            plsc.addupdate_scatter(part_v, [c], g)
        s1 = reduce_partials(sh_l1, lambda i: g1_v[pl.ds(i * 16, 16)])

        for i in range(N_PAD // 16):
            sl = pl.ds(i * 16, 16)
            m = s1[i] * dinv_v[sl]
            h0 = _tanh(m * w10 + b10)
            h1 = _tanh(m * w11 + b11)
            h2 = _tanh(m * w12 + b12)
            h3 = _tanh(m * w13 + b13)
            x2 = h0 * w20 + h1 * w21 + h2 * w22 + h3 * w23
            g2_v[sl] = dinv_v[sl] * x2

        zero_part()
        for j in range(CH_PT):
            r = ed_v[pl.ds(ebase + j * 16, 16)]
            c = ed_v[pl.ds(1536 + ebase + j * 16, 16)]
            g = plsc.load_gather(g2_v, [r])
            plsc.addupdate_scatter(part_v, [c], g)
        agg2 = reduce_partials(sh_l2, lambda i: g2_v[pl.ds(i * 16, 16)])

        for i in range(N_PAD // 16):
            v_v[pl.ds(i * 16, 16)] = _tanh(
                agg2[i] * dinv_v[pl.ds(i * 16, 16)] + b2s)

        wcp.wait()

        def mv_outer(i, accs):
            vvec = v_v[pl.ds(i * 16, 16)]
            nb = i * 16
            for l in range(16):
                s = vvec[l]
                accs = tuple(accs[j] + s * wblk_v[nb + l, pl.ds(OFFS[j], 16)]
                             for j in range(len(OFFS)))
            return accs

        init = tuple(obuf_v[pl.ds(o, 16)] for o in OFFS)
        accs = lax.fori_loop(0, 5, mv_outer, init)
        vtail = v_v[pl.ds(80, 16)]
        for l in range(14):
            s = vtail[l]
            accs = tuple(accs[j] + s * wblk_v[80 + l, pl.ds(OFFS[j], 16)]
                         for j in range(len(OFFS)))
        for j, o in enumerate(OFFS):
            obuf_v[pl.ds(o, 16)] = accs[j]
        pltpu.sync_copy(obuf_v, out_hbm.at[pl.ds(base, COLS_PT)])

    return k(ed, fp, wfc, bfc)


def kernel(feature, edge_index, W1, b1, W2, b2, Wfc, bfc):
    ei = edge_index.astype(jnp.int32)
    ed = jnp.full((2, 1536), 95, jnp.int32).at[:, :1504].set(ei).reshape(-1)
    feat = jnp.zeros((N_PAD,), jnp.float32).at[:94].set(feature[:, 0])
    params = jnp.concatenate([
        W1[0], b1, W2[:, 0], b2, jnp.zeros((3,), jnp.float32)])
    fp = jnp.concatenate([feat, params])
    return _sc_gcn(ed, fp, Wfc, bfc)


# speedup vs baseline: 1.1870x; 1.1870x over previous
import functools

import jax
import jax.numpy as jnp
from jax import lax
from jax.experimental import pallas as pl
from jax.experimental.pallas import tpu as pltpu
from jax.experimental.pallas import tpu_sc as plsc

N_PAD = 96
N_EDGE_CH = 94
NUM_CORES = 1
BLK = 128
N_BLK = 4
OFFS = tuple(range(0, BLK, 16))


def _tanh(x):
    ax = jnp.minimum(jnp.abs(x), 10.0)
    u = 2.0 * ax
    k0 = (u * 1.4426950408889634 + 0.5).astype(jnp.int32)
    kf0 = k0.astype(jnp.float32)
    r0 = (u - kf0 * 0.6931471824645996) - kf0 * (-1.904654323148236e-09)
    adj = (jnp.where(r0 > 0.34657359, 1, 0)
           - jnp.where(r0 < -0.34657359, 1, 0)).astype(jnp.int32)
    k = k0 + adj
    kf = k.astype(jnp.float32)
    r = (u - kf * 0.6931471824645996) - kf * (-1.904654323148236e-09)
    p = jnp.full_like(u, 1.0 / 5040.0)
    for c in (1.0 / 720, 1.0 / 120, 1.0 / 24, 1.0 / 6, 0.5, 1.0, 1.0):
        p = p * r + c
    two_k = lax.bitcast_convert_type(
        lax.shift_left(k + 127, 23), jnp.float32)
    e = p * two_k
    den = e + 1.0
    q = 1.0 / den
    q = q * (2.0 - den * q)
    t = (e - 1.0) * q
    return jnp.sign(x) * t


def _rsqrt(d):
    bits = lax.bitcast_convert_type(d, jnp.int32)
    y = lax.bitcast_convert_type(
        jnp.int32(0x5F3759DF) - (bits >> 1), jnp.float32)
    half = 0.5 * d
    for _ in range(4):
        y = y * (1.5 - half * y * y)
    return y


def _sc_gcn(ed, fp, wfc, bfc):
    mesh = plsc.VectorSubcoreMesh(
        core_axis_name="c", subcore_axis_name="s", num_cores=NUM_CORES)

    @functools.partial(
        pl.kernel,
        mesh=mesh,
        out_type=jax.ShapeDtypeStruct((6400,), jnp.float32),
        compiler_params=pltpu.CompilerParams(needs_layout_passes=False),
        scratch_types=[
            pltpu.VMEM((3008,), jnp.int32),
            pltpu.VMEM((112,), jnp.float32),
            pltpu.VMEM((94, BLK), jnp.float32),
            pltpu.VMEM((94, BLK), jnp.float32),
            pltpu.VMEM((N_BLK * BLK,), jnp.float32),
            pltpu.VMEM((N_PAD,), jnp.float32),
            pltpu.VMEM((N_PAD,), jnp.float32),
            pltpu.VMEM((N_PAD,), jnp.float32),
            pltpu.VMEM((N_PAD,), jnp.float32),
            pltpu.VMEM((N_PAD,), jnp.float32),
            pltpu.VMEM((N_PAD,), jnp.float32),
            pltpu.SemaphoreType.DMA,
            pltpu.SemaphoreType.DMA,
            pltpu.SemaphoreType.DMA,
        ],
    )
    def k(ed_hbm, fp_hbm, wfc_hbm, bfc_hbm, out_hbm,
          ed_v, fp_v, wbufA_v, wbufB_v, obuf_v, dinv_v,
          s1_v, g1_v, g2_v, agg2_v, v_v, wsemA, wsemB, ssem):
        wid = lax.axis_index("s")
        base = ((wid * 50) // 16) * 128

        def wcopy(k, buf, sem):
            return pltpu.make_async_copy(
                wfc_hbm.at[:, pl.ds(base + k * BLK, BLK)], buf, sem)

        wcp0 = wcopy(0, wbufA_v, wsemA)
        wcp1 = wcopy(1, wbufB_v, wsemB)
        wcp0.start()
        wcp1.start()
        cps = [
            pltpu.make_async_copy(ed_hbm, ed_v, ssem),
            pltpu.make_async_copy(fp_hbm, fp_v, ssem),
            pltpu.make_async_copy(
                bfc_hbm.at[pl.ds(base, N_BLK * BLK)], obuf_v, ssem),
        ]
        for cp in cps:
            cp.start()
        for cp in cps:
            cp.wait()

        ones = jnp.ones((16,), jnp.float32)
        for i in range(N_PAD // 16):
            dinv_v[pl.ds(i * 16, 16)] = ones

        def deg_body(e, _):
            c = ed_v[pl.ds(1504 + e * 16, 16)]
            plsc.addupdate_scatter(dinv_v, [c], ones)
            return 0

        lax.fori_loop(0, N_EDGE_CH, deg_body, 0, unroll=4)

        pv = fp_v[pl.ds(96, 16)]
        w10, w11, w12, w13 = pv[0], pv[1], pv[2], pv[3]
        b10, b11, b12, b13 = pv[4], pv[5], pv[6], pv[7]
        w20, w21, w22, w23 = pv[8], pv[9], pv[10], pv[11]
        b2s = pv[12]

        for i in range(N_PAD // 16):
            sl = pl.ds(i * 16, 16)
            di = _rsqrt(dinv_v[sl])
            dinv_v[sl] = di
            g = di * fp_v[sl]
            g1_v[sl] = g
            s1_v[sl] = g

        def edge1_body(e, _):
            r = ed_v[pl.ds(e * 16, 16)]
            c = ed_v[pl.ds(1504 + e * 16, 16)]
            g = plsc.load_gather(g1_v, [r])
            plsc.addupdate_scatter(s1_v, [c], g)
            return 0

        lax.fori_loop(0, N_EDGE_CH, edge1_body, 0, unroll=4)

        for i in range(N_PAD // 16):
            sl = pl.ds(i * 16, 16)
            di = dinv_v[sl]
            m = s1_v[sl] * di
            h0 = _tanh(m * w10 + b10)
            h1 = _tanh(m * w11 + b11)
            h2 = _tanh(m * w12 + b12)
            h3 = _tanh(m * w13 + b13)
            x2 = h0 * w20 + h1 * w21 + h2 * w22 + h3 * w23
            g2 = di * x2
            g2_v[sl] = g2
            agg2_v[sl] = g2

        def edge2_body(e, _):
            r = ed_v[pl.ds(e * 16, 16)]
            c = ed_v[pl.ds(1504 + e * 16, 16)]
            g = plsc.load_gather(g2_v, [r])
            plsc.addupdate_scatter(agg2_v, [c], g)
            return 0

        lax.fori_loop(0, N_EDGE_CH, edge2_body, 0, unroll=4)

        for i in range(N_PAD // 16):
            sl = pl.ds(i * 16, 16)
            v_v[sl] = _tanh(agg2_v[sl] * dinv_v[sl] + b2s)

        def matvec_block(kblk, wbuf):
            ob = kblk * BLK

            def mv_outer(i, accs):
                vvec = v_v[pl.ds(i * 16, 16)]
                nb = i * 16
                for l in range(16):
                    s = vvec[l]
                    accs = tuple(
                        accs[j] + s * wbuf[nb + l, pl.ds(OFFS[j], 16)]
                        for j in range(len(OFFS)))
                return accs

            init = tuple(obuf_v[pl.ds(ob + o, 16)] for o in OFFS)
            accs = lax.fori_loop(0, 5, mv_outer, init)
            vtail = v_v[pl.ds(80, 16)]
            for l in range(14):
                s = vtail[l]
                accs = tuple(
                    accs[j] + s * wbuf[80 + l, pl.ds(OFFS[j], 16)]
                    for j in range(len(OFFS)))
            for j, o in enumerate(OFFS):
                obuf_v[pl.ds(ob + o, 16)] = accs[j]

        wcp0.wait()
        matvec_block(0, wbufA_v)
        wcp2 = wcopy(2, wbufA_v, wsemA)
        wcp2.start()
        wcp1.wait()
        matvec_block(1, wbufB_v)
        wcp3 = wcopy(3, wbufB_v, wsemB)
        wcp3.start()
        wcp2.wait()
        matvec_block(2, wbufA_v)
        wcp3.wait()
        matvec_block(3, wbufB_v)
        pltpu.sync_copy(obuf_v, out_hbm.at[pl.ds(base, N_BLK * BLK)])

    return k(ed, fp, wfc, bfc)


def kernel(feature, edge_index, W1, b1, W2, b2, Wfc, bfc):
    ed = edge_index.astype(jnp.int32).reshape(-1)
    feat = jnp.zeros((N_PAD,), jnp.float32).at[:94].set(feature[:, 0])
    params = jnp.concatenate([
        W1[0], b1, W2[:, 0], b2, jnp.zeros((3,), jnp.float32)])
    fp = jnp.concatenate([feat, params])
    return _sc_gcn(ed, fp, Wfc, bfc)


# speedup vs baseline: 1.2086x; 1.0183x over previous
import functools

import jax
import jax.numpy as jnp
from jax import lax
from jax.experimental import pallas as pl
from jax.experimental.pallas import tpu as pltpu
from jax.experimental.pallas import tpu_sc as plsc

N_PAD = 96
N_EDGE_CH = 94
NUM_CORES = 2
COLS_PT = 256
OFFS = tuple(range(0, COLS_PT, 16))


def _tanh(x):
    ax = jnp.minimum(jnp.abs(x), 10.0)
    u = 2.0 * ax
    k0 = (u * 1.4426950408889634 + 0.5).astype(jnp.int32)
    kf0 = k0.astype(jnp.float32)
    r0 = (u - kf0 * 0.6931471824645996) - kf0 * (-1.904654323148236e-09)
    adj = (jnp.where(r0 > 0.34657359, 1, 0)
           - jnp.where(r0 < -0.34657359, 1, 0)).astype(jnp.int32)
    k = k0 + adj
    kf = k.astype(jnp.float32)
    r = (u - kf * 0.6931471824645996) - kf * (-1.904654323148236e-09)
    p = jnp.full_like(u, 1.0 / 5040.0)
    for c in (1.0 / 720, 1.0 / 120, 1.0 / 24, 1.0 / 6, 0.5, 1.0, 1.0):
        p = p * r + c
    two_k = lax.bitcast_convert_type(
        lax.shift_left(k + 127, 23), jnp.float32)
    e = p * two_k
    den = e + 1.0
    q = 1.0 / den
    q = q * (2.0 - den * q)
    t = (e - 1.0) * q
    return jnp.sign(x) * t


def _rsqrt(d):
    bits = lax.bitcast_convert_type(d, jnp.int32)
    y = lax.bitcast_convert_type(
        jnp.int32(0x5F3759DF) - (bits >> 1), jnp.float32)
    half = 0.5 * d
    for _ in range(4):
        y = y * (1.5 - half * y * y)
    return y


def _sc_gcn(ed, fp, wfc, bfc):
    mesh = plsc.VectorSubcoreMesh(
        core_axis_name="c", subcore_axis_name="s", num_cores=NUM_CORES)

    @functools.partial(
        pl.kernel,
        mesh=mesh,
        out_type=jax.ShapeDtypeStruct((6400,), jnp.float32),
        compiler_params=pltpu.CompilerParams(needs_layout_passes=False),
        scratch_types=[
            pltpu.VMEM((3008,), jnp.int32),
            pltpu.VMEM((112,), jnp.float32),
            pltpu.VMEM((94, COLS_PT), jnp.float32),
            pltpu.VMEM((COLS_PT,), jnp.float32),
            pltpu.VMEM((N_PAD,), jnp.float32),
            pltpu.VMEM((N_PAD,), jnp.float32),
            pltpu.VMEM((N_PAD,), jnp.float32),
            pltpu.VMEM((N_PAD,), jnp.float32),
            pltpu.VMEM((N_PAD,), jnp.float32),
            pltpu.VMEM((N_PAD,), jnp.float32),
            pltpu.SemaphoreType.DMA,
            pltpu.SemaphoreType.DMA,
        ],
    )
    def k(ed_hbm, fp_hbm, wfc_hbm, bfc_hbm, out_hbm,
          ed_v, fp_v, wblk_v, obuf_v, dinv_v,
          s1_v, g1_v, g2_v, agg2_v, v_v, wsem, ssem):
        wid = lax.axis_index("s") * NUM_CORES + lax.axis_index("c")
        base = ((wid * 50) // 32) * 128

        wcp = pltpu.make_async_copy(
            wfc_hbm.at[:, pl.ds(base, COLS_PT)], wblk_v, wsem)
        wcp.start()
        cps = [
            pltpu.make_async_copy(ed_hbm, ed_v, ssem),
            pltpu.make_async_copy(fp_hbm, fp_v, ssem),
            pltpu.make_async_copy(
                bfc_hbm.at[pl.ds(base, COLS_PT)], obuf_v, ssem),
        ]
        for cp in cps:
            cp.start()
        for cp in cps:
            cp.wait()

        ones = jnp.ones((16,), jnp.float32)
        for i in range(N_PAD // 16):
            dinv_v[pl.ds(i * 16, 16)] = ones

        def deg_body(e, _):
            c = ed_v[pl.ds(1504 + e * 16, 16)]
            plsc.addupdate_scatter(dinv_v, [c], ones)
            return 0

        lax.fori_loop(0, N_EDGE_CH, deg_body, 0, unroll=4)

        pv = fp_v[pl.ds(96, 16)]
        w10, w11, w12, w13 = pv[0], pv[1], pv[2], pv[3]
        b10, b11, b12, b13 = pv[4], pv[5], pv[6], pv[7]
        w20, w21, w22, w23 = pv[8], pv[9], pv[10], pv[11]
        b2s = pv[12]

        for i in range(N_PAD // 16):
            sl = pl.ds(i * 16, 16)
            di = _rsqrt(dinv_v[sl])
            dinv_v[sl] = di
            g = di * fp_v[sl]
            g1_v[sl] = g
            s1_v[sl] = g

        def edge1_body(e, _):
            r = ed_v[pl.ds(e * 16, 16)]
            c = ed_v[pl.ds(1504 + e * 16, 16)]
            g = plsc.load_gather(g1_v, [r])
            plsc.addupdate_scatter(s1_v, [c], g)
            return 0

        lax.fori_loop(0, N_EDGE_CH, edge1_body, 0, unroll=4)

        for i in range(N_PAD // 16):
            sl = pl.ds(i * 16, 16)
            di = dinv_v[sl]
            m = s1_v[sl] * di
            h0 = _tanh(m * w10 + b10)
            h1 = _tanh(m * w11 + b11)
            h2 = _tanh(m * w12 + b12)
            h3 = _tanh(m * w13 + b13)
            x2 = h0 * w20 + h1 * w21 + h2 * w22 + h3 * w23
            g2 = di * x2
            g2_v[sl] = g2
            agg2_v[sl] = g2

        def edge2_body(e, _):
            r = ed_v[pl.ds(e * 16, 16)]
            c = ed_v[pl.ds(1504 + e * 16, 16)]
            g = plsc.load_gather(g2_v, [r])
            plsc.addupdate_scatter(agg2_v, [c], g)
            return 0

        lax.fori_loop(0, N_EDGE_CH, edge2_body, 0, unroll=4)

        for i in range(N_PAD // 16):
            sl = pl.ds(i * 16, 16)
            v_v[sl] = _tanh(agg2_v[sl] * dinv_v[sl] + b2s)

        wcp.wait()

        def mv_outer(i, accs):
            vvec = v_v[pl.ds(i * 16, 16)]
            nb = i * 16
            for l in range(16):
                s = vvec[l]
                accs = tuple(accs[j] + s * wblk_v[nb + l, pl.ds(OFFS[j], 16)]
                             for j in range(len(OFFS)))
            return accs

        init = tuple(obuf_v[pl.ds(o, 16)] for o in OFFS)
        accs = lax.fori_loop(0, 5, mv_outer, init)
        vtail = v_v[pl.ds(80, 16)]
        for l in range(14):
            s = vtail[l]
            accs = tuple(accs[j] + s * wblk_v[80 + l, pl.ds(OFFS[j], 16)]
                         for j in range(len(OFFS)))
        for j, o in enumerate(OFFS):
            obuf_v[pl.ds(o, 16)] = accs[j]
        pltpu.sync_copy(obuf_v, out_hbm.at[pl.ds(base, COLS_PT)])

    return k(ed, fp, wfc, bfc)


def kernel(feature, edge_index, W1, b1, W2, b2, Wfc, bfc):
    ed = edge_index.astype(jnp.int32).reshape(-1)
    feat = jnp.zeros((N_PAD,), jnp.float32).at[:94].set(feature[:, 0])
    params = jnp.concatenate([
        W1[0], b1, W2[:, 0], b2, jnp.zeros((3,), jnp.float32)])
    fp = jnp.concatenate([feat, params])
    return _sc_gcn(ed, fp, Wfc, bfc)
